# baseline (speedup 1.0000x reference)
import functools

import jax
import jax.numpy as jnp
from jax import lax
from jax.experimental import pallas as pl
from jax.experimental.pallas import tpu as pltpu
from jax.experimental.pallas import tpu_sc as plsc

BATCH = 16384
EMBED_DIM = 64
AVG_RATING = 3.0

_NC = 2
_NS = 16
_NW = _NC * _NS
_BPW = BATCH // _NW
_CHUNK = 128
_NCHUNK = _BPW // _CHUNK
_GROUPS = _BPW // 16


def _body(user_hbm, item_hbm, user_emb_hbm, item_emb_hbm,
          user_bias_hbm, item_bias_hbm,
          out_hbm, ub_hbm, ib_hbm,
          idx_u, idx_i, u_rows, i_rows, ub_v, ib_v, out_v, sem):
    wid = lax.axis_index("s") * _NC + lax.axis_index("c")
    base = wid * _BPW

    for k in range(_NCHUNK):
        pltpu.sync_copy(user_hbm.at[pl.ds(base + k * _CHUNK, _CHUNK)], idx_u.at[k])
        pltpu.sync_copy(item_hbm.at[pl.ds(base + k * _CHUNK, _CHUNK)], idx_i.at[k])

    copies = []
    for k in range(_NCHUNK):
        sl = pl.ds(k * _CHUNK, _CHUNK)
        copies.append(pltpu.async_copy(user_emb_hbm.at[idx_u.at[k]], u_rows.at[sl], sem))
        copies.append(pltpu.async_copy(item_emb_hbm.at[idx_i.at[k]], i_rows.at[sl], sem))
        copies.append(pltpu.async_copy(user_bias_hbm.at[idx_u.at[k]], ub_v.at[sl], sem))
        copies.append(pltpu.async_copy(item_bias_hbm.at[idx_i.at[k]], ib_v.at[sl], sem))
    for c in copies:
        c.wait()

    lane = lax.iota(jnp.int32, 16)

    def group(g, carry):
        vecs = []
        for j in range(16):
            r = g * 16 + j
            acc = u_rows[r, pl.ds(0, 16)] * i_rows[r, pl.ds(0, 16)]
            for t in range(1, EMBED_DIM // 16):
                sl = pl.ds(t * 16, 16)
                acc = acc + u_rows[r, sl] * i_rows[r, sl]
            vecs.append(acc)
        sh = 1
        while len(vecs) > 1:
            idx = lane ^ sh
            m = (lane & sh) != 0
            nxt = []
            for k in range(len(vecs) // 2):
                u, v = vecs[2 * k], vecs[2 * k + 1]
                gu = u.at[idx].get(mode="promise_in_bounds")
                gv = v.at[idx].get(mode="promise_in_bounds")
                nxt.append(jnp.where(m, v + gv, u + gu))
            vecs = nxt
            sh *= 2
        sl = pl.ds(g * 16, 16)
        out_v[sl] = AVG_RATING + ub_v[sl] + ib_v[sl] + vecs[0]
        return carry

    lax.fori_loop(0, _GROUPS, group, 0)

    pltpu.sync_copy(out_v, out_hbm.at[pl.ds(base, _BPW)])
    pltpu.sync_copy(ub_v, ub_hbm.at[pl.ds(base, _BPW)])
    pltpu.sync_copy(ib_v, ib_hbm.at[pl.ds(base, _BPW)])


@functools.partial(
    pl.kernel,
    mesh=plsc.VectorSubcoreMesh(core_axis_name="c", subcore_axis_name="s"),
    compiler_params=pltpu.CompilerParams(use_tc_tiling_on_sc=False),
    out_type=(
        jax.ShapeDtypeStruct((BATCH,), jnp.float32),
        jax.ShapeDtypeStruct((BATCH,), jnp.float32),
        jax.ShapeDtypeStruct((BATCH,), jnp.float32),
    ),
    scratch_types=[
        pltpu.VMEM((_NCHUNK, _CHUNK), jnp.int32),
        pltpu.VMEM((_NCHUNK, _CHUNK), jnp.int32),
        pltpu.VMEM((_BPW, EMBED_DIM), jnp.float32),
        pltpu.VMEM((_BPW, EMBED_DIM), jnp.float32),
        pltpu.VMEM((_BPW,), jnp.float32),
        pltpu.VMEM((_BPW,), jnp.float32),
        pltpu.VMEM((_BPW,), jnp.float32),
        pltpu.SemaphoreType.DMA,
    ],
)
def _svd_sc(*refs):
    _body(*refs)


def kernel(user, item, user_emb, item_emb, user_bias, item_bias):
    return _svd_sc(user, item, user_emb, item_emb, user_bias, item_bias)
